# baseline (speedup 1.0000x reference)
import functools

import jax
import jax.numpy as jnp
import numpy as np
from jax import lax
from jax.experimental import pallas as pl
from jax.experimental.pallas import tpu as pltpu
from jax.experimental.pallas import tpu_sc as plsc

N_NODES = 100000
D = 128
B = 20000
S = 10

NW = 32
BP = 20480
BPW = BP // NW
A0 = 256
A1 = 1024
MAXA = max(A0, A1)
C = 16
NCH = BPW // C
NV = D // 16
DW = D // 2
NVW = DW // 16

NBUF = 4


def _sc_body(nodes_hbm, posf_hbm, negf_hbm, feat_hbm,
             self_out, aggp_out, aggn_out,
             idxs_v, idxp_v, idxn_v, bufs, semgs, semos):
    c = lax.axis_index("c")
    s_ax = lax.axis_index("s")
    wbase = jnp.where(c == 0, s_ax * A0, 16 * A0 + s_ax * A1)
    nch = jnp.where(c == 0, A0 // C, A1 // C)

    pltpu.sync_copy(nodes_hbm.at[pl.ds(pl.multiple_of(wbase, 8), MAXA)],
                    idxs_v)
    pltpu.sync_copy(posf_hbm.at[pl.ds(pl.multiple_of(wbase * S, 8), MAXA * S)],
                    idxp_v)
    pltpu.sync_copy(negf_hbm.at[pl.ds(pl.multiple_of(wbase * S, 8), MAXA * S)],
                    idxn_v)

    def issue_g(g, b):
        selfr, posr, negr, _, _ = bufs[b]
        off = pl.multiple_of(g * C, 8)
        ioff = pl.multiple_of(g * C * S, 8)
        pltpu.async_copy(feat_hbm.at[idxs_v.at[pl.ds(off, C)]], selfr, semgs[b])
        pltpu.async_copy(feat_hbm.at[idxp_v.at[pl.ds(ioff, C * S)]], posr,
                         semgs[b])
        pltpu.async_copy(feat_hbm.at[idxn_v.at[pl.ds(ioff, C * S)]], negr,
                         semgs[b])

    def wait_g(b):
        selfr, posr, negr, _, _ = bufs[b]
        pltpu.make_async_copy(feat_hbm.at[idxs_v.at[pl.ds(0, C)]],
                              selfr, semgs[b]).wait()
        pltpu.make_async_copy(feat_hbm.at[idxp_v.at[pl.ds(0, C * S)]],
                              posr, semgs[b]).wait()
        pltpu.make_async_copy(feat_hbm.at[idxn_v.at[pl.ds(0, C * S)]],
                              negr, semgs[b]).wait()

    def wait_out(b):
        _, _, _, aggp, aggn = bufs[b]
        row = pl.ds(0, C)
        pltpu.make_async_copy(aggp, aggp_out.at[row], semos[b]).wait()
        pltpu.make_async_copy(aggn, aggn_out.at[row], semos[b]).wait()

    def step(g, b, bnext, p):
        selfr, posr, negr, aggp, aggn = bufs[b]
        row0 = pl.multiple_of(wbase + g * C, 8)
        wait_g(b)
        pltpu.sync_copy(selfr, self_out.at[pl.ds(row0, C)])

        @pl.when(p > 0)
        def _():
            wait_out(b)

        def unpack(ref, row, sl):
            w = ref[row, sl]
            lo = lax.bitcast_convert_type(lax.shift_left(w, 16), jnp.float32)
            hi = lax.bitcast_convert_type(lax.bitwise_and(w, jnp.int32(-65536)),
                                          jnp.float32)
            return lo, hi

        def pack(lo, hi):
            wlo = lax.bitcast_convert_type(lo, jnp.int32) + jnp.int32(0x8000)
            whi = lax.bitcast_convert_type(hi, jnp.int32) + jnp.int32(0x8000)
            return lax.bitwise_or(
                lax.shift_right_logical(wlo, 16),
                lax.bitwise_and(whi, jnp.int32(-65536)))

        def seed(i, carry):
            row = i * S
            for v in range(NVW):
                sl = pl.ds(v * 16, 16)
                pe, po = unpack(posr, row, sl)
                ne, no = unpack(negr, row, sl)
                for s in range(1, S):
                    pe2, po2 = unpack(posr, row + s, sl)
                    ne2, no2 = unpack(negr, row + s, sl)
                    pe, po = pe + pe2, po + po2
                    ne, no = ne + ne2, no + no2
                aggp[i, sl] = pack(pe * (1.0 / S), po * (1.0 / S))
                aggn[i, sl] = pack(ne * (1.0 / S), no * (1.0 / S))
            return carry

        lax.fori_loop(0, C, seed, 0)
        pltpu.async_copy(aggp, aggp_out.at[pl.ds(row0, C)], semos[b])
        pltpu.async_copy(aggn, aggn_out.at[pl.ds(row0, C)], semos[b])

        @pl.when(g + NBUF - 1 < nch)
        def _():
            issue_g(g + NBUF - 1, bnext)

    for j in range(NBUF - 1):
        issue_g(j, j)

    P = nch // NBUF

    def group(p, carry):
        for j in range(NBUF):
            step(p * NBUF + j, j, (j - 1) % NBUF, p)
        return carry

    lax.fori_loop(0, P, group, 0)
    for b in range(NBUF):
        wait_out(b)


_sc_gather_agg = functools.partial(
    pl.kernel,
    out_type=[jax.ShapeDtypeStruct((BP, DW), jnp.int32)] * 3,
    mesh=plsc.VectorSubcoreMesh(core_axis_name="c", subcore_axis_name="s"),
    compiler_params=pltpu.CompilerParams(use_tc_tiling_on_sc=False),
    scratch_types=[
        pltpu.VMEM((MAXA,), jnp.int32),
        pltpu.VMEM((MAXA * S,), jnp.int32),
        pltpu.VMEM((MAXA * S,), jnp.int32),
        tuple(
            (pltpu.VMEM((C, DW), jnp.int32),
             pltpu.VMEM((C * S, DW), jnp.int32),
             pltpu.VMEM((C * S, DW), jnp.int32),
             pltpu.VMEM((C, DW), jnp.int32),
             pltpu.VMEM((C, DW), jnp.int32))
            for _ in range(NBUF)),
        tuple(pltpu.SemaphoreType.DMA for _ in range(NBUF)),
        tuple(pltpu.SemaphoreType.DMA for _ in range(NBUF)),
    ],
)(_sc_body)


def _unpack_cols(x):
    lo = lax.bitcast_convert_type(lax.shift_left(x, 16), jnp.float32)
    hi = lax.bitcast_convert_type(lax.bitwise_and(x, jnp.int32(-65536)),
                                  jnp.float32)
    return jnp.concatenate([lo, hi], axis=1)


def _tc_body(self_ref, aggp_ref, aggn_ref, wb_ref, wu_ref, ob_ref, ou_ref):
    s = _unpack_cols(self_ref[...])
    ap = _unpack_cols(aggp_ref[...])
    an = _unpack_cols(aggn_ref[...])
    wb = wb_ref[...]
    wu = wu_ref[...]
    bal = (jnp.dot(s, wb[:D], preferred_element_type=jnp.float32)
           + jnp.dot(ap, wb[D:], preferred_element_type=jnp.float32))
    unbal = (jnp.dot(s, wu[:D], preferred_element_type=jnp.float32)
             + jnp.dot(an, wu[D:], preferred_element_type=jnp.float32))
    ob_ref[...] = jnp.tanh(bal)
    ou_ref[...] = jnp.tanh(unbal)


_TC_BS = 2048


def _tc_encode(selff, aggp, aggn, W_bal, W_unbal):
    grid = BP // _TC_BS
    in_spec = pl.BlockSpec((_TC_BS, DW), lambda i: (i, 0))
    w_spec = pl.BlockSpec((2 * D, D), lambda i: (0, 0))
    out_spec = pl.BlockSpec((_TC_BS, D), lambda i: (i, 0))
    return pl.pallas_call(
        _tc_body,
        grid=(grid,),
        in_specs=[in_spec, in_spec, in_spec, w_spec, w_spec],
        out_specs=[out_spec, out_spec],
        out_shape=[jax.ShapeDtypeStruct((BP, D), jnp.float32)] * 2,
    )(selff, aggp, aggn, W_bal, W_unbal)


def kernel(nodes, neigh_pos, neigh_neg, features, W_bal, W_unbal):
    nodes_p = jnp.pad(nodes, (0, BP - B))
    posf = jnp.pad(neigh_pos.reshape(-1), (0, (BP - B) * S))
    negf = jnp.pad(neigh_neg.reshape(-1), (0, (BP - B) * S))
    lo = lax.bitcast_convert_type(
        features[:, :DW].astype(jnp.bfloat16), jnp.uint16).astype(jnp.int32)
    hi = lax.bitcast_convert_type(
        features[:, DW:].astype(jnp.bfloat16), jnp.uint16).astype(jnp.int32)
    tabp = lax.bitwise_or(lo, lax.shift_left(hi, 16))
    selfp, aggp, aggn = _sc_gather_agg(nodes_p, posf, negf, tabp)
    ob, ou = _tc_encode(selfp, aggp, aggn, W_bal, W_unbal)
    return ob[:B], ou[:B]


# speedup vs baseline: 1.0778x; 1.0778x over previous
import functools

import jax
import jax.numpy as jnp
import numpy as np
from jax import lax
from jax.experimental import pallas as pl
from jax.experimental.pallas import tpu as pltpu
from jax.experimental.pallas import tpu_sc as plsc

N_NODES = 100000
D = 128
B = 20000
S = 10

NW = 32
BP = 20480
BPW = BP // NW
A0 = BPW
A1 = BPW
MAXA = max(A0, A1)
C = 16
NCH = BPW // C
NV = D // 16
DW = D // 2
NVW = DW // 16

NBUF = 4


def _sc_body(nodes_hbm, posf_hbm, negf_hbm, feat_hbm,
             self_out, aggp_out, aggn_out,
             idxs_v, idxp_v, idxn_v, bufs, semgs, semos):
    c = lax.axis_index("c")
    s_ax = lax.axis_index("s")
    wbase = jnp.where(c == 0, s_ax * A0, 16 * A0 + s_ax * A1)
    nch = jnp.where(c == 0, A0 // C, A1 // C)

    pltpu.sync_copy(nodes_hbm.at[pl.ds(pl.multiple_of(wbase, 8), MAXA)],
                    idxs_v)
    pltpu.sync_copy(posf_hbm.at[pl.ds(pl.multiple_of(wbase * S, 8), MAXA * S)],
                    idxp_v)
    pltpu.sync_copy(negf_hbm.at[pl.ds(pl.multiple_of(wbase * S, 8), MAXA * S)],
                    idxn_v)

    def issue_g(g, b):
        selfr, posr, negr, _, _ = bufs[b]
        off = pl.multiple_of(g * C, 8)
        ioff = pl.multiple_of(g * C * S, 8)
        pltpu.async_copy(feat_hbm.at[idxs_v.at[pl.ds(off, C)]], selfr, semgs[b])
        pltpu.async_copy(feat_hbm.at[idxp_v.at[pl.ds(ioff, C * S)]], posr,
                         semgs[b])
        pltpu.async_copy(feat_hbm.at[idxn_v.at[pl.ds(ioff, C * S)]], negr,
                         semgs[b])

    def wait_g(b):
        selfr, posr, negr, _, _ = bufs[b]
        pltpu.make_async_copy(feat_hbm.at[idxs_v.at[pl.ds(0, C)]],
                              selfr, semgs[b]).wait()
        pltpu.make_async_copy(feat_hbm.at[idxp_v.at[pl.ds(0, C * S)]],
                              posr, semgs[b]).wait()
        pltpu.make_async_copy(feat_hbm.at[idxn_v.at[pl.ds(0, C * S)]],
                              negr, semgs[b]).wait()

    def wait_out(b):
        _, _, _, aggp, aggn = bufs[b]
        row = pl.ds(0, C)
        pltpu.make_async_copy(aggp, aggp_out.at[row], semos[b]).wait()
        pltpu.make_async_copy(aggn, aggn_out.at[row], semos[b]).wait()

    def step(g, b, bnext, p):
        selfr, posr, negr, aggp, aggn = bufs[b]
        row0 = pl.multiple_of(wbase + g * C, 8)
        wait_g(b)
        pltpu.sync_copy(selfr, self_out.at[pl.ds(row0, C)])

        @pl.when(p > 0)
        def _():
            wait_out(b)

        def unpack(ref, row, sl):
            w = ref[row, sl]
            lo = lax.bitcast_convert_type(lax.shift_left(w, 16), jnp.float32)
            hi = lax.bitcast_convert_type(lax.bitwise_and(w, jnp.int32(-65536)),
                                          jnp.float32)
            return lo, hi

        def pack(lo, hi):
            wlo = lax.bitcast_convert_type(lo, jnp.int32) + jnp.int32(0x8000)
            whi = lax.bitcast_convert_type(hi, jnp.int32) + jnp.int32(0x8000)
            return lax.bitwise_or(
                lax.shift_right_logical(wlo, 16),
                lax.bitwise_and(whi, jnp.int32(-65536)))

        def seed(i, carry):
            row = i * S
            for v in range(NVW):
                sl = pl.ds(v * 16, 16)
                pe, po = unpack(posr, row, sl)
                ne, no = unpack(negr, row, sl)
                for s in range(1, S):
                    pe2, po2 = unpack(posr, row + s, sl)
                    ne2, no2 = unpack(negr, row + s, sl)
                    pe, po = pe + pe2, po + po2
                    ne, no = ne + ne2, no + no2
                aggp[i, sl] = pack(pe * (1.0 / S), po * (1.0 / S))
                aggn[i, sl] = pack(ne * (1.0 / S), no * (1.0 / S))
            return carry

        lax.fori_loop(0, C, seed, 0)
        pltpu.async_copy(aggp, aggp_out.at[pl.ds(row0, C)], semos[b])
        pltpu.async_copy(aggn, aggn_out.at[pl.ds(row0, C)], semos[b])

        @pl.when(g + NBUF - 1 < nch)
        def _():
            issue_g(g + NBUF - 1, bnext)

    for j in range(NBUF - 1):
        issue_g(j, j)

    P = nch // NBUF

    def group(p, carry):
        for j in range(NBUF):
            step(p * NBUF + j, j, (j - 1) % NBUF, p)
        return carry

    lax.fori_loop(0, P, group, 0)
    for b in range(NBUF):
        wait_out(b)


_sc_gather_agg = functools.partial(
    pl.kernel,
    out_type=[jax.ShapeDtypeStruct((BP, DW), jnp.int32)] * 3,
    mesh=plsc.VectorSubcoreMesh(core_axis_name="c", subcore_axis_name="s"),
    compiler_params=pltpu.CompilerParams(use_tc_tiling_on_sc=False),
    scratch_types=[
        pltpu.VMEM((MAXA,), jnp.int32),
        pltpu.VMEM((MAXA * S,), jnp.int32),
        pltpu.VMEM((MAXA * S,), jnp.int32),
        tuple(
            (pltpu.VMEM((C, DW), jnp.int32),
             pltpu.VMEM((C * S, DW), jnp.int32),
             pltpu.VMEM((C * S, DW), jnp.int32),
             pltpu.VMEM((C, DW), jnp.int32),
             pltpu.VMEM((C, DW), jnp.int32))
            for _ in range(NBUF)),
        tuple(pltpu.SemaphoreType.DMA for _ in range(NBUF)),
        tuple(pltpu.SemaphoreType.DMA for _ in range(NBUF)),
    ],
)(_sc_body)


def _unpack_cols(x):
    lo = lax.bitcast_convert_type(lax.shift_left(x, 16), jnp.float32)
    hi = lax.bitcast_convert_type(lax.bitwise_and(x, jnp.int32(-65536)),
                                  jnp.float32)
    return jnp.concatenate([lo, hi], axis=1)


def _tc_body(self_ref, aggp_ref, aggn_ref, wb_ref, wu_ref, ob_ref, ou_ref):
    s = _unpack_cols(self_ref[...])
    ap = _unpack_cols(aggp_ref[...])
    an = _unpack_cols(aggn_ref[...])
    wb = wb_ref[...]
    wu = wu_ref[...]
    bal = (jnp.dot(s, wb[:D], preferred_element_type=jnp.float32)
           + jnp.dot(ap, wb[D:], preferred_element_type=jnp.float32))
    unbal = (jnp.dot(s, wu[:D], preferred_element_type=jnp.float32)
             + jnp.dot(an, wu[D:], preferred_element_type=jnp.float32))
    ob_ref[...] = jnp.tanh(bal)
    ou_ref[...] = jnp.tanh(unbal)


_TC_BS = 2048


def _tc_encode(selff, aggp, aggn, W_bal, W_unbal):
    bs = B // 10
    in_spec = pl.BlockSpec((bs, DW), lambda i: (i, 0))
    w_spec = pl.BlockSpec((2 * D, D), lambda i: (0, 0))
    out_spec = pl.BlockSpec((bs, D), lambda i: (i, 0))
    return pl.pallas_call(
        _tc_body,
        grid=(10,),
        in_specs=[in_spec, in_spec, in_spec, w_spec, w_spec],
        out_specs=[out_spec, out_spec],
        out_shape=[jax.ShapeDtypeStruct((B, D), jnp.float32)] * 2,
    )(selff, aggp, aggn, W_bal, W_unbal)


def kernel(nodes, neigh_pos, neigh_neg, features, W_bal, W_unbal):
    nodes_p = jnp.pad(nodes, (0, BP - B))
    posf = jnp.pad(neigh_pos.reshape(-1), (0, (BP - B) * S))
    negf = jnp.pad(neigh_neg.reshape(-1), (0, (BP - B) * S))
    lo = lax.bitcast_convert_type(
        features[:, :DW].astype(jnp.bfloat16), jnp.uint16).astype(jnp.int32)
    hi = lax.bitcast_convert_type(
        features[:, DW:].astype(jnp.bfloat16), jnp.uint16).astype(jnp.int32)
    tabp = lax.bitwise_or(lo, lax.shift_left(hi, 16))
    selfp, aggp, aggn = _sc_gather_agg(nodes_p, posf, negf, tabp)
    ob, ou = _tc_encode(selfp, aggp, aggn, W_bal, W_unbal)
    return ob, ou


# speedup vs baseline: 1.0796x; 1.0016x over previous
import functools

import jax
import jax.numpy as jnp
import numpy as np
from jax import lax
from jax.experimental import pallas as pl
from jax.experimental.pallas import tpu as pltpu
from jax.experimental.pallas import tpu_sc as plsc

N_NODES = 100000
D = 128
B = 20000
S = 10

NW = 32
BP = 20480
BPW = BP // NW
A0 = BPW
A1 = BPW
MAXA = max(A0, A1)
C = 16
NCH = BPW // C
NV = D // 16
DW = D // 2
NVW = DW // 16

NBUF = 4


def _sc_body(nodes_hbm, posf_hbm, negf_hbm, feat_hbm,
             self_out, aggp_out, aggn_out,
             idxs_v, idxp_v, idxn_v, bufs, semgs, semos, *, apw):
    wid = lax.axis_index("s") * 2 + lax.axis_index("c")
    wbase = wid * apw
    nch = apw // C

    pltpu.sync_copy(nodes_hbm.at[pl.ds(pl.multiple_of(wbase, 8), apw)],
                    idxs_v)
    pltpu.sync_copy(posf_hbm.at[pl.ds(pl.multiple_of(wbase * S, 8), apw * S)],
                    idxp_v)
    pltpu.sync_copy(negf_hbm.at[pl.ds(pl.multiple_of(wbase * S, 8), apw * S)],
                    idxn_v)

    def issue_g(g, b):
        selfr, posr, negr, _, _ = bufs[b]
        off = pl.multiple_of(g * C, 8)
        ioff = pl.multiple_of(g * C * S, 8)
        pltpu.async_copy(feat_hbm.at[idxs_v.at[pl.ds(off, C)]], selfr, semgs[b])
        pltpu.async_copy(feat_hbm.at[idxp_v.at[pl.ds(ioff, C * S)]], posr,
                         semgs[b])
        pltpu.async_copy(feat_hbm.at[idxn_v.at[pl.ds(ioff, C * S)]], negr,
                         semgs[b])

    def wait_g(b):
        selfr, posr, negr, _, _ = bufs[b]
        pltpu.make_async_copy(feat_hbm.at[idxs_v.at[pl.ds(0, C)]],
                              selfr, semgs[b]).wait()
        pltpu.make_async_copy(feat_hbm.at[idxp_v.at[pl.ds(0, C * S)]],
                              posr, semgs[b]).wait()
        pltpu.make_async_copy(feat_hbm.at[idxn_v.at[pl.ds(0, C * S)]],
                              negr, semgs[b]).wait()

    def wait_out(b):
        _, _, _, aggp, aggn = bufs[b]
        row = pl.ds(0, C)
        pltpu.make_async_copy(aggp, aggp_out.at[row], semos[b]).wait()
        pltpu.make_async_copy(aggn, aggn_out.at[row], semos[b]).wait()

    def step(g, b, bnext, p):
        selfr, posr, negr, aggp, aggn = bufs[b]
        row0 = pl.multiple_of(wbase + g * C, 8)
        wait_g(b)
        pltpu.sync_copy(selfr, self_out.at[pl.ds(row0, C)])

        @pl.when(p > 0)
        def _():
            wait_out(b)

        def unpack(ref, row, sl):
            w = ref[row, sl]
            lo = lax.bitcast_convert_type(lax.shift_left(w, 16), jnp.float32)
            hi = lax.bitcast_convert_type(lax.bitwise_and(w, jnp.int32(-65536)),
                                          jnp.float32)
            return lo, hi

        def pack(lo, hi):
            wlo = lax.bitcast_convert_type(lo, jnp.int32) + jnp.int32(0x8000)
            whi = lax.bitcast_convert_type(hi, jnp.int32) + jnp.int32(0x8000)
            return lax.bitwise_or(
                lax.shift_right_logical(wlo, 16),
                lax.bitwise_and(whi, jnp.int32(-65536)))

        def seed(i, carry):
            row = i * S
            for v in range(NVW):
                sl = pl.ds(v * 16, 16)
                pe, po = unpack(posr, row, sl)
                ne, no = unpack(negr, row, sl)
                for s in range(1, S):
                    pe2, po2 = unpack(posr, row + s, sl)
                    ne2, no2 = unpack(negr, row + s, sl)
                    pe, po = pe + pe2, po + po2
                    ne, no = ne + ne2, no + no2
                aggp[i, sl] = pack(pe * (1.0 / S), po * (1.0 / S))
                aggn[i, sl] = pack(ne * (1.0 / S), no * (1.0 / S))
            return carry

        lax.fori_loop(0, C, seed, 0)
        pltpu.async_copy(aggp, aggp_out.at[pl.ds(row0, C)], semos[b])
        pltpu.async_copy(aggn, aggn_out.at[pl.ds(row0, C)], semos[b])

        @pl.when(g + NBUF - 1 < nch)
        def _():
            issue_g(g + NBUF - 1, bnext)

    for j in range(NBUF - 1):
        issue_g(j, j)

    P = nch // NBUF

    def group(p, carry):
        for j in range(NBUF):
            step(p * NBUF + j, j, (j - 1) % NBUF, p)
        return carry

    lax.fori_loop(0, P, group, 0)
    for b in range(NBUF):
        wait_out(b)


def _make_sc_kernel(rows):
    apw = rows // NW
    body = functools.partial(_sc_body, apw=apw)
    return functools.partial(
        pl.kernel,
        out_type=[jax.ShapeDtypeStruct((rows, DW), jnp.int32)] * 3,
        mesh=plsc.VectorSubcoreMesh(core_axis_name="c", subcore_axis_name="s"),
        compiler_params=pltpu.CompilerParams(use_tc_tiling_on_sc=False),
        scratch_types=[
            pltpu.VMEM((apw,), jnp.int32),
            pltpu.VMEM((apw * S,), jnp.int32),
            pltpu.VMEM((apw * S,), jnp.int32),
            tuple(
                (pltpu.VMEM((C, DW), jnp.int32),
                 pltpu.VMEM((C * S, DW), jnp.int32),
                 pltpu.VMEM((C * S, DW), jnp.int32),
                 pltpu.VMEM((C, DW), jnp.int32),
                 pltpu.VMEM((C, DW), jnp.int32))
                for _ in range(NBUF)),
            tuple(pltpu.SemaphoreType.DMA for _ in range(NBUF)),
            tuple(pltpu.SemaphoreType.DMA for _ in range(NBUF)),
        ],
    )(body)


HALF = BP // 2
_sc_gather_half = _make_sc_kernel(HALF)


def _unpack_cols(x):
    lo = lax.bitcast_convert_type(lax.shift_left(x, 16), jnp.float32)
    hi = lax.bitcast_convert_type(lax.bitwise_and(x, jnp.int32(-65536)),
                                  jnp.float32)
    return jnp.concatenate([lo, hi], axis=1)


def _tc_body(self_ref, aggp_ref, aggn_ref, wb_ref, wu_ref, ob_ref, ou_ref):
    s = _unpack_cols(self_ref[...])
    ap = _unpack_cols(aggp_ref[...])
    an = _unpack_cols(aggn_ref[...])
    wb = wb_ref[...]
    wu = wu_ref[...]
    bal = (jnp.dot(s, wb[:D], preferred_element_type=jnp.float32)
           + jnp.dot(ap, wb[D:], preferred_element_type=jnp.float32))
    unbal = (jnp.dot(s, wu[:D], preferred_element_type=jnp.float32)
             + jnp.dot(an, wu[D:], preferred_element_type=jnp.float32))
    ob_ref[...] = jnp.tanh(bal)
    ou_ref[...] = jnp.tanh(unbal)


_TC_BS = 2048


def _tc_encode(selff, aggp, aggn, W_bal, W_unbal, rows_out):
    bs = rows_out // 5
    in_spec = pl.BlockSpec((bs, DW), lambda i: (i, 0))
    w_spec = pl.BlockSpec((2 * D, D), lambda i: (0, 0))
    out_spec = pl.BlockSpec((bs, D), lambda i: (i, 0))
    return pl.pallas_call(
        _tc_body,
        grid=(5,),
        in_specs=[in_spec, in_spec, in_spec, w_spec, w_spec],
        out_specs=[out_spec, out_spec],
        out_shape=[jax.ShapeDtypeStruct((rows_out, D), jnp.float32)] * 2,
    )(selff, aggp, aggn, W_bal, W_unbal)


def kernel(nodes, neigh_pos, neigh_neg, features, W_bal, W_unbal):
    nodes_p = jnp.pad(nodes, (0, BP - B))
    posf = jnp.pad(neigh_pos.reshape(-1), (0, (BP - B) * S))
    negf = jnp.pad(neigh_neg.reshape(-1), (0, (BP - B) * S))
    lo = lax.bitcast_convert_type(
        features[:, :DW].astype(jnp.bfloat16), jnp.uint16).astype(jnp.int32)
    hi = lax.bitcast_convert_type(
        features[:, DW:].astype(jnp.bfloat16), jnp.uint16).astype(jnp.int32)
    tabp = lax.bitwise_or(lo, lax.shift_left(hi, 16))
    h = HALF
    s1, p1, n1 = _sc_gather_half(nodes_p[:h], posf[:h * S], negf[:h * S],
                                 tabp)
    s2, p2, n2 = _sc_gather_half(nodes_p[h:], posf[h * S:], negf[h * S:],
                                 tabp)
    ob1, ou1 = _tc_encode(s1, p1, n1, W_bal, W_unbal, h)
    ob2, ou2 = _tc_encode(s2, p2, n2, W_bal, W_unbal, B - h)
    return (jnp.concatenate([ob1, ob2]), jnp.concatenate([ou1, ou2]))
